# Initial kernel scaffold; baseline (speedup 1.0000x reference)
import jax
import jax.numpy as jnp
from jax.experimental import pallas as pl


def kernel(node_q, node_p, edges, observations, kalman_gain):
    raise NotImplementedError("write your pallas kernel here")


# speedup vs baseline: 6.5394x; 6.5394x over previous
import functools

import jax
import jax.numpy as jnp
from jax import lax
from jax.experimental import pallas as pl
from jax.experimental.pallas import tpu as pltpu
from jax.experimental.pallas import tpu_sc as plsc

N = 10000
D = 128
E = 320000
DT = 0.01

NC = 2
NS = 16
NW = NC * NS
EPW = E // NW
CHUNK = 80
NCHUNKS = EPW // CHUNK
ZR = 1000
NZT = N // ZR
BB = 40
NBB = ZR // BB
GW = 16
NPAD = 10240
GZR = NPAD // NZT
GB = 64
NGB = GZR // GB
PR = GZR // 8
PB = 64


def _sc_scatter(src, dst, q, zrow):
    mesh = plsc.VectorSubcoreMesh(core_axis_name="c", subcore_axis_name="s")

    @functools.partial(
        pl.kernel,
        out_type=(
            jax.ShapeDtypeStruct((NC, N, D), jnp.float32),
            jax.ShapeDtypeStruct((NC, NPAD // 8, D), jnp.float32),
        ),
        mesh=mesh,
        scratch_types=[
            pltpu.VMEM((CHUNK,), jnp.int32),
            pltpu.VMEM((CHUNK,), jnp.int32),
            pltpu.VMEM((CHUNK, D), jnp.float32),
            pltpu.VMEM((CHUNK, GW), jnp.float32),
            pltpu.VMEM((GB, GW), jnp.float32),
            pltpu.VMEM((PB, D), jnp.float32),
            pltpu.VMEM_SHARED((N, D), jnp.float32),
            pltpu.VMEM_SHARED((NPAD, GW), jnp.float32),
            pltpu.SemaphoreType.DMA,
        ],
        compiler_params=pltpu.CompilerParams(use_tc_tiling_on_sc=False),
    )
    def k(src_hbm, dst_hbm, q_hbm, zrow_hbm,
          s_out, g_out, src_v, dst_v, rows_v, ones_v, bg_v, big_v,
          s_sh, g_sh, sem):
        c = lax.axis_index("c")
        s = lax.axis_index("s")
        r0 = s * ZR
        g0 = s * GZR

        one16 = jnp.full((GW,), 1.0, dtype=jnp.float32)
        zero16 = jnp.zeros((GW,), dtype=jnp.float32)
        for r in range(CHUNK):
            ones_v[r, :] = one16
        for r in range(GB):
            bg_v[r, :] = zero16

        @pl.when(s < NZT)
        def _():
            pltpu.sync_copy(zrow_hbm, big_v.at[pl.ds(0, BB)])
            for j in range(NBB):
                pltpu.sync_copy(big_v.at[pl.ds(0, BB)],
                                s_sh.at[pl.ds(r0 + j * BB, BB)])
            for j in range(NGB):
                pltpu.sync_copy(bg_v, g_sh.at[pl.ds(g0 + j * GB, GB)])

        plsc.subcore_barrier()

        base = (c * NS + s) * EPW

        def body(i, carry):
            e0 = base + i * CHUNK
            pltpu.sync_copy(src_hbm.at[pl.ds(e0, CHUNK)], src_v)
            pltpu.sync_copy(dst_hbm.at[pl.ds(e0, CHUNK)], dst_v)
            pltpu.async_copy(q_hbm.at[src_v], rows_v, sem).wait()
            pltpu.sync_copy(rows_v, s_sh.at[dst_v], add=True)
            pltpu.sync_copy(ones_v, g_sh.at[dst_v], add=True)
            return carry

        lax.fori_loop(0, NCHUNKS, body, 0)
        plsc.subcore_barrier()

        @pl.when(s < NZT)
        def _():
            for j in range(NBB):
                rj = r0 + j * BB
                pltpu.sync_copy(s_sh.at[pl.ds(rj, BB)], big_v.at[pl.ds(0, BB)])
                pltpu.sync_copy(big_v.at[pl.ds(0, BB)],
                                s_out.at[c, pl.ds(rj, BB)])

            for h in range(GZR // (PB * 8)):

                def pack(j, carry):
                    pltpu.sync_copy(
                        g_sh.at[pl.ds(g0 + h * PB * 8 + j * GB, GB)], bg_v)
                    for r in range(GB):
                        big_v[j * (GB // 8) + r // 8,
                              pl.ds(GW * (r % 8), GW)] = bg_v[r, :]
                    return carry

                lax.fori_loop(0, (PB * 8) // GB, pack, 0)
                pltpu.sync_copy(
                    big_v, g_out.at[c, pl.ds(s * PR + h * PB, PB)])

    return k(src, dst, q, zrow)


BLK = 1000


def _finish_body(q_ref, p_ref, s_ref, g_ref, obs_ref, gain_ref,
                 qo_ref, po_ref):
    q = q_ref[...]
    p = p_ref[...]
    s = s_ref[0] + s_ref[1]
    deg = g_ref[0, :, 0:1] + g_ref[1, :, 0:1]
    msg = deg * q - s
    q_new = q + DT * p + (0.5 * DT * DT) * msg
    p_new = p + DT * msg

    @pl.when(pl.program_id(0) == 0)
    def _():
        innov = obs_ref[...] - q_new
        rows = lax.broadcasted_iota(jnp.int32, q_new.shape, 0)
        m = rows == 0
        qo_ref[...] = jnp.where(m, q_new + gain_ref[0:1] * innov, q_new)
        po_ref[...] = jnp.where(m, p_new + gain_ref[1:2] * innov, p_new)

    @pl.when(pl.program_id(0) != 0)
    def _():
        qo_ref[...] = q_new
        po_ref[...] = p_new


def _finish(q, p, s_parts, g_parts, obs, gain2):
    grid = (N // BLK,)
    return pl.pallas_call(
        _finish_body,
        grid=grid,
        in_specs=[
            pl.BlockSpec((BLK, D), lambda i: (i, 0)),
            pl.BlockSpec((BLK, D), lambda i: (i, 0)),
            pl.BlockSpec((NC, BLK, D), lambda i: (0, i, 0)),
            pl.BlockSpec((NC, BLK, GW), lambda i: (0, i, 0)),
            pl.BlockSpec((1, D), lambda i: (0, 0)),
            pl.BlockSpec((2, D), lambda i: (0, 0)),
        ],
        out_specs=[
            pl.BlockSpec((BLK, D), lambda i: (i, 0)),
            pl.BlockSpec((BLK, D), lambda i: (i, 0)),
        ],
        out_shape=[
            jax.ShapeDtypeStruct((N, D), jnp.float32),
            jax.ShapeDtypeStruct((N, D), jnp.float32),
        ],
    )(q, p, s_parts, g_parts, obs, gain2)


def kernel(node_q, node_p, edges, observations, kalman_gain):
    q = node_q.reshape(N, D)
    p = node_p.reshape(N, D)
    src = edges[:, 0]
    dst = edges[:, 1]
    zrow = jnp.zeros((BB, D), jnp.float32)
    s_parts, g_packed = _sc_scatter(src, dst, q, zrow)
    g_parts = g_packed.reshape(NC, NPAD, GW)[:, :N, :]
    obs = observations.reshape(1, D)
    gain2 = kalman_gain.reshape(2, D)
    qo, po = _finish(q, p, s_parts, g_parts, obs, gain2)
    return qo.reshape(node_q.shape), po.reshape(node_p.shape)


# speedup vs baseline: 7.7783x; 1.1894x over previous
import functools

import jax
import jax.numpy as jnp
from jax import lax
from jax.experimental import pallas as pl
from jax.experimental.pallas import tpu as pltpu
from jax.experimental.pallas import tpu_sc as plsc

N = 10000
D = 128
E = 320000
DT = 0.01

NC = 2
NS = 16
NW = NC * NS
EPW = E // NW
CHUNK = 80
NCHUNKS = EPW // CHUNK
ZR = 1000
NZT = N // ZR
BB = 40
NBB = ZR // BB
GW = 16
NPAD = 10240
GZR = NPAD // NZT
GB = 64
NGB = GZR // GB
PR = GZR // 8
PB = 64


def _sc_scatter(src, dst, q, zrow):
    mesh = plsc.VectorSubcoreMesh(core_axis_name="c", subcore_axis_name="s")

    @functools.partial(
        pl.kernel,
        out_type=(
            jax.ShapeDtypeStruct((NC, N, D), jnp.float32),
            jax.ShapeDtypeStruct((NC, NPAD // 8, D), jnp.float32),
        ),
        mesh=mesh,
        scratch_types=[
            pltpu.VMEM((CHUNK,), jnp.int32),
            pltpu.VMEM((CHUNK,), jnp.int32),
            pltpu.VMEM((CHUNK,), jnp.int32),
            pltpu.VMEM((CHUNK, D), jnp.float32),
            pltpu.VMEM((CHUNK, D), jnp.float32),
            pltpu.VMEM((CHUNK, GW), jnp.float32),
            pltpu.VMEM((GB, GW), jnp.float32),
            pltpu.VMEM((PB, D), jnp.float32),
            pltpu.VMEM_SHARED((N, D), jnp.float32),
            pltpu.VMEM_SHARED((NPAD, GW), jnp.float32),
            pltpu.SemaphoreType.DMA,
            pltpu.SemaphoreType.DMA,
            pltpu.SemaphoreType.DMA,
        ],
        compiler_params=pltpu.CompilerParams(use_tc_tiling_on_sc=False),
    )
    def k(src_hbm, dst_hbm, q_hbm, zrow_hbm,
          s_out, g_out, src_v, dst0_v, dst1_v, rows0_v, rows1_v,
          ones_v, bg_v, big_v, s_sh, g_sh, gsem, ssem0, ssem1):
        c = lax.axis_index("c")
        s = lax.axis_index("s")
        r0 = s * ZR
        g0 = s * GZR

        one16 = jnp.full((GW,), 1.0, dtype=jnp.float32)
        zero16 = jnp.zeros((GW,), dtype=jnp.float32)
        for r in range(CHUNK):
            ones_v[r, :] = one16
        for r in range(GB):
            bg_v[r, :] = zero16

        @pl.when(s < NZT)
        def _():
            pltpu.sync_copy(zrow_hbm, big_v.at[pl.ds(0, BB)])
            for j in range(NBB):
                pltpu.sync_copy(big_v.at[pl.ds(0, BB)],
                                s_sh.at[pl.ds(r0 + j * BB, BB)])
            for j in range(NGB):
                pltpu.sync_copy(bg_v, g_sh.at[pl.ds(g0 + j * GB, GB)])

        plsc.subcore_barrier()

        base = (c * NS + s) * EPW

        def copy_idx(k, dref):
            e0 = base + k * CHUNK
            pltpu.sync_copy(src_hbm.at[pl.ds(e0, CHUNK)], src_v)
            pltpu.sync_copy(dst_hbm.at[pl.ds(e0, CHUNK)], dref)

        def start_gather(rref):
            pltpu.async_copy(q_hbm.at[src_v], rref, gsem)

        def wait_gather(rref):
            pltpu.make_async_copy(q_hbm.at[src_v], rref, gsem).wait()

        def start_scatter(rref, dref, sem_):
            pltpu.async_copy(rref, s_sh.at[dref], sem_, add=True)
            pltpu.async_copy(ones_v, g_sh.at[dref], sem_, add=True)

        def wait_scatter(rref, dref, sem_):
            pltpu.make_async_copy(rref, s_sh.at[dref], sem_).wait()
            pltpu.make_async_copy(ones_v, g_sh.at[dref], sem_).wait()

        copy_idx(0, dst0_v)
        start_gather(rows0_v)

        def step(t, carry):
            k0 = 2 * t
            wait_gather(rows0_v)
            start_scatter(rows0_v, dst0_v, ssem0)

            @pl.when(t > 0)
            def _():
                wait_scatter(rows1_v, dst1_v, ssem1)

            copy_idx(k0 + 1, dst1_v)
            start_gather(rows1_v)

            wait_gather(rows1_v)
            start_scatter(rows1_v, dst1_v, ssem1)
            wait_scatter(rows0_v, dst0_v, ssem0)
            copy_idx(k0 + 2, dst0_v)
            start_gather(rows0_v)
            return carry

        lax.fori_loop(0, NCHUNKS // 2, step, 0)

        wait_gather(rows0_v)
        start_scatter(rows0_v, dst0_v, ssem0)
        wait_scatter(rows1_v, dst1_v, ssem1)
        wait_scatter(rows0_v, dst0_v, ssem0)
        plsc.subcore_barrier()

        @pl.when(s < NZT)
        def _():
            for j in range(NBB):
                rj = r0 + j * BB
                pltpu.sync_copy(s_sh.at[pl.ds(rj, BB)], big_v.at[pl.ds(0, BB)])
                pltpu.sync_copy(big_v.at[pl.ds(0, BB)],
                                s_out.at[c, pl.ds(rj, BB)])

            for h in range(GZR // (PB * 8)):

                def pack(j, carry):
                    pltpu.sync_copy(
                        g_sh.at[pl.ds(g0 + h * PB * 8 + j * GB, GB)], bg_v)
                    for r in range(GB):
                        big_v[j * (GB // 8) + r // 8,
                              pl.ds(GW * (r % 8), GW)] = bg_v[r, :]
                    return carry

                lax.fori_loop(0, (PB * 8) // GB, pack, 0)
                pltpu.sync_copy(
                    big_v, g_out.at[c, pl.ds(s * PR + h * PB, PB)])

    return k(src, dst, q, zrow)


BLK = 1000


def _finish_body(q_ref, p_ref, s_ref, g_ref, obs_ref, gain_ref,
                 qo_ref, po_ref):
    q = q_ref[...]
    p = p_ref[...]
    s = s_ref[0] + s_ref[1]
    deg = g_ref[0, :, 0:1] + g_ref[1, :, 0:1]
    msg = deg * q - s
    q_new = q + DT * p + (0.5 * DT * DT) * msg
    p_new = p + DT * msg

    @pl.when(pl.program_id(0) == 0)
    def _():
        innov = obs_ref[...] - q_new
        rows = lax.broadcasted_iota(jnp.int32, q_new.shape, 0)
        m = rows == 0
        qo_ref[...] = jnp.where(m, q_new + gain_ref[0:1] * innov, q_new)
        po_ref[...] = jnp.where(m, p_new + gain_ref[1:2] * innov, p_new)

    @pl.when(pl.program_id(0) != 0)
    def _():
        qo_ref[...] = q_new
        po_ref[...] = p_new


def _finish(q, p, s_parts, g_parts, obs, gain2):
    grid = (N // BLK,)
    return pl.pallas_call(
        _finish_body,
        grid=grid,
        in_specs=[
            pl.BlockSpec((BLK, D), lambda i: (i, 0)),
            pl.BlockSpec((BLK, D), lambda i: (i, 0)),
            pl.BlockSpec((NC, BLK, D), lambda i: (0, i, 0)),
            pl.BlockSpec((NC, BLK, GW), lambda i: (0, i, 0)),
            pl.BlockSpec((1, D), lambda i: (0, 0)),
            pl.BlockSpec((2, D), lambda i: (0, 0)),
        ],
        out_specs=[
            pl.BlockSpec((BLK, D), lambda i: (i, 0)),
            pl.BlockSpec((BLK, D), lambda i: (i, 0)),
        ],
        out_shape=[
            jax.ShapeDtypeStruct((N, D), jnp.float32),
            jax.ShapeDtypeStruct((N, D), jnp.float32),
        ],
    )(q, p, s_parts, g_parts, obs, gain2)


def kernel(node_q, node_p, edges, observations, kalman_gain):
    q = node_q.reshape(N, D)
    p = node_p.reshape(N, D)
    src = edges[:, 0]
    dst = edges[:, 1]
    zrow = jnp.zeros((BB, D), jnp.float32)
    s_parts, g_packed = _sc_scatter(src, dst, q, zrow)
    g_parts = g_packed.reshape(NC, NPAD, GW)[:, :N, :]
    obs = observations.reshape(1, D)
    gain2 = kalman_gain.reshape(2, D)
    qo, po = _finish(q, p, s_parts, g_parts, obs, gain2)
    return qo.reshape(node_q.shape), po.reshape(node_p.shape)


# speedup vs baseline: 8.0948x; 1.0407x over previous
import functools

import jax
import jax.numpy as jnp
from jax import lax
from jax.experimental import pallas as pl
from jax.experimental.pallas import tpu as pltpu
from jax.experimental.pallas import tpu_sc as plsc

N = 10000
D = 128
E = 320000
DT = 0.01

NC = 2
NS = 16
NW = NC * NS
EPW = E // NW
CHUNK = 80
NCHUNKS = EPW // CHUNK
ZR = 1000
NZT = N // ZR
BB = 40
NBB = ZR // BB
GW = 16
NPAD = 10240
GZR = NPAD // NZT
GB = 64
NGB = GZR // GB
PR = GZR // 8
PB = 64


def _sc_scatter(src, dst, q, zrow):
    mesh = plsc.VectorSubcoreMesh(core_axis_name="c", subcore_axis_name="s")

    @functools.partial(
        pl.kernel,
        out_type=(
            jax.ShapeDtypeStruct((NC, N, D), jnp.bfloat16),
            jax.ShapeDtypeStruct((NC, NPAD // 8, D), jnp.float32),
        ),
        mesh=mesh,
        scratch_types=[
            pltpu.VMEM((CHUNK,), jnp.int32),
            pltpu.VMEM((CHUNK,), jnp.int32),
            pltpu.VMEM((CHUNK,), jnp.int32),
            pltpu.VMEM((CHUNK, D), jnp.bfloat16),
            pltpu.VMEM((CHUNK, D), jnp.bfloat16),
            pltpu.VMEM((CHUNK, GW), jnp.float32),
            pltpu.VMEM((GB, GW), jnp.float32),
            pltpu.VMEM((PB, D), jnp.float32),
            pltpu.VMEM((BB, D), jnp.bfloat16),
            pltpu.VMEM_SHARED((N, D), jnp.bfloat16),
            pltpu.VMEM_SHARED((NPAD, GW), jnp.float32),
            pltpu.SemaphoreType.DMA,
            pltpu.SemaphoreType.DMA,
            pltpu.SemaphoreType.DMA,
        ],
        compiler_params=pltpu.CompilerParams(use_tc_tiling_on_sc=False),
    )
    def k(src_hbm, dst_hbm, q_hbm, zrow_hbm,
          s_out, g_out, src_v, dst0_v, dst1_v, rows0_v, rows1_v,
          ones_v, bg_v, big_v, sbb_v, s_sh, g_sh, gsem, ssem0, ssem1):
        c = lax.axis_index("c")
        s = lax.axis_index("s")
        r0 = s * ZR
        g0 = s * GZR

        one16 = jnp.full((GW,), 1.0, dtype=jnp.float32)
        zero16 = jnp.zeros((GW,), dtype=jnp.float32)
        for r in range(CHUNK):
            ones_v[r, :] = one16
        for r in range(GB):
            bg_v[r, :] = zero16

        @pl.when(s < NZT)
        def _():
            pltpu.sync_copy(zrow_hbm, sbb_v)
            for j in range(NBB):
                pltpu.sync_copy(sbb_v, s_sh.at[pl.ds(r0 + j * BB, BB)])
            for j in range(NGB):
                pltpu.sync_copy(bg_v, g_sh.at[pl.ds(g0 + j * GB, GB)])

        plsc.subcore_barrier()

        base = (c * NS + s) * EPW

        def copy_idx(k, dref):
            e0 = base + k * CHUNK
            pltpu.sync_copy(src_hbm.at[pl.ds(e0, CHUNK)], src_v)
            pltpu.sync_copy(dst_hbm.at[pl.ds(e0, CHUNK)], dref)

        def start_gather(rref):
            pltpu.async_copy(q_hbm.at[src_v], rref, gsem)

        def wait_gather(rref):
            pltpu.make_async_copy(q_hbm.at[src_v], rref, gsem).wait()

        def start_scatter(rref, dref, sem_):
            pltpu.async_copy(rref, s_sh.at[dref], sem_, add=True)
            pltpu.async_copy(ones_v, g_sh.at[dref], sem_, add=True)

        def wait_scatter(rref, dref, sem_):
            pltpu.make_async_copy(rref, s_sh.at[dref], sem_).wait()
            pltpu.make_async_copy(ones_v, g_sh.at[dref], sem_).wait()

        copy_idx(0, dst0_v)
        start_gather(rows0_v)

        def step(t, carry):
            k0 = 2 * t
            wait_gather(rows0_v)
            start_scatter(rows0_v, dst0_v, ssem0)

            @pl.when(t > 0)
            def _():
                wait_scatter(rows1_v, dst1_v, ssem1)

            copy_idx(k0 + 1, dst1_v)
            start_gather(rows1_v)

            wait_gather(rows1_v)
            start_scatter(rows1_v, dst1_v, ssem1)
            wait_scatter(rows0_v, dst0_v, ssem0)
            copy_idx(k0 + 2, dst0_v)
            start_gather(rows0_v)
            return carry

        lax.fori_loop(0, NCHUNKS // 2, step, 0)

        wait_gather(rows0_v)
        start_scatter(rows0_v, dst0_v, ssem0)
        wait_scatter(rows1_v, dst1_v, ssem1)
        wait_scatter(rows0_v, dst0_v, ssem0)
        plsc.subcore_barrier()

        @pl.when(s < NZT)
        def _():
            for j in range(NBB):
                rj = r0 + j * BB
                pltpu.sync_copy(s_sh.at[pl.ds(rj, BB)], sbb_v)
                pltpu.sync_copy(sbb_v, s_out.at[c, pl.ds(rj, BB)])

            for h in range(GZR // (PB * 8)):

                def pack(j, carry):
                    pltpu.sync_copy(
                        g_sh.at[pl.ds(g0 + h * PB * 8 + j * GB, GB)], bg_v)
                    for r in range(GB):
                        big_v[j * (GB // 8) + r // 8,
                              pl.ds(GW * (r % 8), GW)] = bg_v[r, :]
                    return carry

                lax.fori_loop(0, (PB * 8) // GB, pack, 0)
                pltpu.sync_copy(
                    big_v, g_out.at[c, pl.ds(s * PR + h * PB, PB)])

    return k(src, dst, q, zrow)


BLK = 1000


def _finish_body(q_ref, p_ref, s_ref, g_ref, obs_ref, gain_ref,
                 qo_ref, po_ref):
    q = q_ref[...]
    p = p_ref[...]
    s = (s_ref[0].astype(jnp.float32) + s_ref[1].astype(jnp.float32))
    deg = g_ref[0, :, 0:1] + g_ref[1, :, 0:1]
    msg = deg * q - s
    q_new = q + DT * p + (0.5 * DT * DT) * msg
    p_new = p + DT * msg

    @pl.when(pl.program_id(0) == 0)
    def _():
        innov = obs_ref[...] - q_new
        rows = lax.broadcasted_iota(jnp.int32, q_new.shape, 0)
        m = rows == 0
        qo_ref[...] = jnp.where(m, q_new + gain_ref[0:1] * innov, q_new)
        po_ref[...] = jnp.where(m, p_new + gain_ref[1:2] * innov, p_new)

    @pl.when(pl.program_id(0) != 0)
    def _():
        qo_ref[...] = q_new
        po_ref[...] = p_new


def _finish(q, p, s_parts, g_parts, obs, gain2):
    grid = (N // BLK,)
    return pl.pallas_call(
        _finish_body,
        grid=grid,
        in_specs=[
            pl.BlockSpec((BLK, D), lambda i: (i, 0)),
            pl.BlockSpec((BLK, D), lambda i: (i, 0)),
            pl.BlockSpec((NC, BLK, D), lambda i: (0, i, 0)),
            pl.BlockSpec((NC, BLK, GW), lambda i: (0, i, 0)),
            pl.BlockSpec((1, D), lambda i: (0, 0)),
            pl.BlockSpec((2, D), lambda i: (0, 0)),
        ],
        out_specs=[
            pl.BlockSpec((BLK, D), lambda i: (i, 0)),
            pl.BlockSpec((BLK, D), lambda i: (i, 0)),
        ],
        out_shape=[
            jax.ShapeDtypeStruct((N, D), jnp.float32),
            jax.ShapeDtypeStruct((N, D), jnp.float32),
        ],
    )(q, p, s_parts, g_parts, obs, gain2)


def kernel(node_q, node_p, edges, observations, kalman_gain):
    q = node_q.reshape(N, D)
    p = node_p.reshape(N, D)
    src = edges[:, 0]
    dst = edges[:, 1]
    zrow = jnp.zeros((BB, D), jnp.bfloat16)
    s_parts, g_packed = _sc_scatter(src, dst, q.astype(jnp.bfloat16), zrow)
    g_parts = g_packed.reshape(NC, NPAD, GW)[:, :N, :]
    obs = observations.reshape(1, D)
    gain2 = kalman_gain.reshape(2, D)
    qo, po = _finish(q, p, s_parts, g_parts, obs, gain2)
    return qo.reshape(node_q.shape), po.reshape(node_p.shape)


# speedup vs baseline: 13.9606x; 1.7246x over previous
import functools

import jax
import jax.numpy as jnp
from jax import lax
from jax.experimental import pallas as pl
from jax.experimental.pallas import tpu as pltpu
from jax.experimental.pallas import tpu_sc as plsc

N = 10000
D = 128
E = 320000
DT = 0.01

NC = 2
NS = 16
NW = NC * NS
EPW = E // NW
CHUNK = 400
NCHUNKS = EPW // CHUNK
ZR = 1000
NZT = N // ZR
BB = 40
NBB = ZR // BB
GW = 16
NPAD = 10240
GZR = NPAD // NZT
GB = 64
NGB = GZR // GB
PR = GZR // 8
PB = 64


def _sc_scatter(src, dst, q, zrow):
    mesh = plsc.VectorSubcoreMesh(core_axis_name="c", subcore_axis_name="s")

    @functools.partial(
        pl.kernel,
        out_type=(
            jax.ShapeDtypeStruct((NC, N, D), jnp.bfloat16),
            jax.ShapeDtypeStruct((NC, NPAD // 8, D), jnp.float32),
        ),
        mesh=mesh,
        scratch_types=[
            pltpu.VMEM((CHUNK,), jnp.int32),
            pltpu.VMEM((CHUNK,), jnp.int32),
            pltpu.VMEM((CHUNK,), jnp.int32),
            pltpu.VMEM((CHUNK, D), jnp.bfloat16),
            pltpu.VMEM((CHUNK, D), jnp.bfloat16),
            pltpu.VMEM((CHUNK, GW), jnp.float32),
            pltpu.VMEM((GB, GW), jnp.float32),
            pltpu.VMEM((PB, D), jnp.float32),
            pltpu.VMEM((BB, D), jnp.bfloat16),
            pltpu.VMEM_SHARED((N, D), jnp.bfloat16),
            pltpu.VMEM_SHARED((NPAD, GW), jnp.float32),
            pltpu.SemaphoreType.DMA,
            pltpu.SemaphoreType.DMA,
            pltpu.SemaphoreType.DMA,
        ],
        compiler_params=pltpu.CompilerParams(use_tc_tiling_on_sc=False),
    )
    def k(src_hbm, dst_hbm, q_hbm, zrow_hbm,
          s_out, g_out, src_v, dst0_v, dst1_v, rows0_v, rows1_v,
          ones_v, bg_v, big_v, sbb_v, s_sh, g_sh, gsem, ssem0, ssem1):
        c = lax.axis_index("c")
        s = lax.axis_index("s")
        r0 = s * ZR
        g0 = s * GZR

        one16 = jnp.full((GW,), 1.0, dtype=jnp.float32)
        zero16 = jnp.zeros((GW,), dtype=jnp.float32)
        for r in range(CHUNK):
            ones_v[r, :] = one16
        for r in range(GB):
            bg_v[r, :] = zero16

        @pl.when(s < NZT)
        def _():
            pltpu.sync_copy(zrow_hbm, sbb_v)
            for j in range(NBB):
                pltpu.sync_copy(sbb_v, s_sh.at[pl.ds(r0 + j * BB, BB)])
            for j in range(NGB):
                pltpu.sync_copy(bg_v, g_sh.at[pl.ds(g0 + j * GB, GB)])

        plsc.subcore_barrier()

        base = (c * NS + s) * EPW

        def copy_idx(k, dref):
            e0 = base + k * CHUNK
            pltpu.sync_copy(src_hbm.at[pl.ds(e0, CHUNK)], src_v)
            pltpu.sync_copy(dst_hbm.at[pl.ds(e0, CHUNK)], dref)

        def start_gather(rref):
            pltpu.async_copy(q_hbm.at[src_v], rref, gsem)

        def wait_gather(rref):
            pltpu.make_async_copy(q_hbm.at[src_v], rref, gsem).wait()

        def start_scatter(rref, dref, sem_):
            pltpu.async_copy(rref, s_sh.at[dref], sem_, add=True)
            pltpu.async_copy(ones_v, g_sh.at[dref], sem_, add=True)

        def wait_scatter(rref, dref, sem_):
            pltpu.make_async_copy(rref, s_sh.at[dref], sem_).wait()
            pltpu.make_async_copy(ones_v, g_sh.at[dref], sem_).wait()

        copy_idx(0, dst0_v)
        start_gather(rows0_v)

        def step(t, carry):
            k0 = 2 * t
            wait_gather(rows0_v)
            start_scatter(rows0_v, dst0_v, ssem0)

            @pl.when(t > 0)
            def _():
                wait_scatter(rows1_v, dst1_v, ssem1)

            copy_idx(k0 + 1, dst1_v)
            start_gather(rows1_v)

            wait_gather(rows1_v)
            start_scatter(rows1_v, dst1_v, ssem1)
            wait_scatter(rows0_v, dst0_v, ssem0)
            copy_idx(k0 + 2, dst0_v)
            start_gather(rows0_v)
            return carry

        lax.fori_loop(0, NCHUNKS // 2, step, 0)

        wait_gather(rows0_v)
        start_scatter(rows0_v, dst0_v, ssem0)
        wait_scatter(rows1_v, dst1_v, ssem1)
        wait_scatter(rows0_v, dst0_v, ssem0)
        plsc.subcore_barrier()

        @pl.when(s < NZT)
        def _():
            for j in range(NBB):
                rj = r0 + j * BB
                pltpu.sync_copy(s_sh.at[pl.ds(rj, BB)], sbb_v)
                pltpu.sync_copy(sbb_v, s_out.at[c, pl.ds(rj, BB)])

            for h in range(GZR // (PB * 8)):

                def pack(j, carry):
                    pltpu.sync_copy(
                        g_sh.at[pl.ds(g0 + h * PB * 8 + j * GB, GB)], bg_v)
                    for r in range(GB):
                        big_v[j * (GB // 8) + r // 8,
                              pl.ds(GW * (r % 8), GW)] = bg_v[r, :]
                    return carry

                lax.fori_loop(0, (PB * 8) // GB, pack, 0)
                pltpu.sync_copy(
                    big_v, g_out.at[c, pl.ds(s * PR + h * PB, PB)])

    return k(src, dst, q, zrow)


BLK = 1000


def _finish_body(q_ref, p_ref, s_ref, g_ref, obs_ref, gain_ref,
                 qo_ref, po_ref):
    q = q_ref[...]
    p = p_ref[...]
    s = (s_ref[0].astype(jnp.float32) + s_ref[1].astype(jnp.float32))
    deg = g_ref[0, :, 0:1] + g_ref[1, :, 0:1]
    msg = deg * q - s
    q_new = q + DT * p + (0.5 * DT * DT) * msg
    p_new = p + DT * msg

    @pl.when(pl.program_id(0) == 0)
    def _():
        innov = obs_ref[...] - q_new
        rows = lax.broadcasted_iota(jnp.int32, q_new.shape, 0)
        m = rows == 0
        qo_ref[...] = jnp.where(m, q_new + gain_ref[0:1] * innov, q_new)
        po_ref[...] = jnp.where(m, p_new + gain_ref[1:2] * innov, p_new)

    @pl.when(pl.program_id(0) != 0)
    def _():
        qo_ref[...] = q_new
        po_ref[...] = p_new


def _finish(q, p, s_parts, g_parts, obs, gain2):
    grid = (N // BLK,)
    return pl.pallas_call(
        _finish_body,
        grid=grid,
        in_specs=[
            pl.BlockSpec((BLK, D), lambda i: (i, 0)),
            pl.BlockSpec((BLK, D), lambda i: (i, 0)),
            pl.BlockSpec((NC, BLK, D), lambda i: (0, i, 0)),
            pl.BlockSpec((NC, BLK, GW), lambda i: (0, i, 0)),
            pl.BlockSpec((1, D), lambda i: (0, 0)),
            pl.BlockSpec((2, D), lambda i: (0, 0)),
        ],
        out_specs=[
            pl.BlockSpec((BLK, D), lambda i: (i, 0)),
            pl.BlockSpec((BLK, D), lambda i: (i, 0)),
        ],
        out_shape=[
            jax.ShapeDtypeStruct((N, D), jnp.float32),
            jax.ShapeDtypeStruct((N, D), jnp.float32),
        ],
    )(q, p, s_parts, g_parts, obs, gain2)


def kernel(node_q, node_p, edges, observations, kalman_gain):
    q = node_q.reshape(N, D)
    p = node_p.reshape(N, D)
    src = edges[:, 0]
    dst = edges[:, 1]
    zrow = jnp.zeros((BB, D), jnp.bfloat16)
    s_parts, g_packed = _sc_scatter(src, dst, q.astype(jnp.bfloat16), zrow)
    g_parts = g_packed.reshape(NC, NPAD, GW)[:, :N, :]
    obs = observations.reshape(1, D)
    gain2 = kalman_gain.reshape(2, D)
    qo, po = _finish(q, p, s_parts, g_parts, obs, gain2)
    return qo.reshape(node_q.shape), po.reshape(node_p.shape)


# speedup vs baseline: 15.5623x; 1.1147x over previous
import functools

import jax
import jax.numpy as jnp
from jax import lax
from jax.experimental import pallas as pl
from jax.experimental.pallas import tpu as pltpu
from jax.experimental.pallas import tpu_sc as plsc

N = 10000
D = 128
E = 320000
DT = 0.01

NC = 2
NS = 16
NW = NC * NS
EPW = E // NW
CHUNK = 400
NCHUNKS = EPW // CHUNK
ZR = 1000
NZT = N // ZR
BB = 40
NBB = ZR // BB
GW = 16
NPAD = 10240
GZR = NPAD // NZT
GB = 64
NGB = GZR // GB
PR = GZR // 8
PB = 64


def _sc_scatter(src, dst, q):
    mesh = plsc.VectorSubcoreMesh(core_axis_name="c", subcore_axis_name="s")

    @functools.partial(
        pl.kernel,
        out_type=(
            jax.ShapeDtypeStruct((NC, N, D), jnp.bfloat16),
            jax.ShapeDtypeStruct((NC, NPAD, GW), jnp.float32),
        ),
        mesh=mesh,
        scratch_types=[
            pltpu.VMEM((CHUNK,), jnp.int32),
            pltpu.VMEM((CHUNK,), jnp.int32),
            pltpu.VMEM((CHUNK,), jnp.int32),
            pltpu.VMEM((CHUNK, D), jnp.bfloat16),
            pltpu.VMEM((CHUNK, D), jnp.bfloat16),
            pltpu.VMEM((CHUNK, GW), jnp.float32),
            pltpu.VMEM((GB, GW), jnp.float32),
            pltpu.VMEM((BB, D), jnp.bfloat16),
            pltpu.VMEM_SHARED((N, D), jnp.bfloat16),
            pltpu.VMEM_SHARED((NPAD, GW), jnp.float32),
            pltpu.SemaphoreType.DMA,
            pltpu.SemaphoreType.DMA,
            pltpu.SemaphoreType.DMA,
        ],
        compiler_params=pltpu.CompilerParams(use_tc_tiling_on_sc=False),
    )
    def k(src_hbm, dst_hbm, q_hbm,
          s_out, g_out, src_v, dst0_v, dst1_v, rows0_v, rows1_v,
          ones_v, zg_v, zb_v, s_sh, g_sh, gsem, ssem0, ssem1):
        c = lax.axis_index("c")
        s = lax.axis_index("s")
        r0 = s * ZR
        g0 = s * GZR

        one16 = jnp.full((GW,), 1.0, dtype=jnp.float32)
        zero16 = jnp.zeros((GW,), dtype=jnp.float32)
        zero32b = jnp.zeros((32,), dtype=jnp.bfloat16)
        for r in range(CHUNK):
            ones_v[r, :] = one16
        for r in range(GB):
            zg_v[r, :] = zero16
        for r in range(BB):
            for cc in range(D // 32):
                zb_v[r, pl.ds(cc * 32, 32)] = zero32b

        @pl.when(s < NZT)
        def _():
            for j in range(NBB):
                pltpu.sync_copy(zb_v, s_sh.at[pl.ds(r0 + j * BB, BB)])
            for j in range(NGB):
                pltpu.sync_copy(zg_v, g_sh.at[pl.ds(g0 + j * GB, GB)])

        plsc.subcore_barrier()

        base = (c * NS + s) * EPW

        def copy_idx(k, dref):
            e0 = base + k * CHUNK
            pltpu.sync_copy(src_hbm.at[pl.ds(e0, CHUNK)], src_v)
            pltpu.sync_copy(dst_hbm.at[pl.ds(e0, CHUNK)], dref)

        def start_gather(rref):
            pltpu.async_copy(q_hbm.at[src_v], rref, gsem)

        def wait_gather(rref):
            pltpu.make_async_copy(q_hbm.at[src_v], rref, gsem).wait()

        def start_scatter(rref, dref, sem_):
            pltpu.async_copy(rref, s_sh.at[dref], sem_, add=True)
            pltpu.async_copy(ones_v, g_sh.at[dref], sem_, add=True)

        def wait_scatter(rref, dref, sem_):
            pltpu.make_async_copy(rref, s_sh.at[dref], sem_).wait()
            pltpu.make_async_copy(ones_v, g_sh.at[dref], sem_).wait()

        copy_idx(0, dst0_v)
        start_gather(rows0_v)

        def step(t, carry):
            k0 = 2 * t
            wait_gather(rows0_v)
            start_scatter(rows0_v, dst0_v, ssem0)

            @pl.when(t > 0)
            def _():
                wait_scatter(rows1_v, dst1_v, ssem1)

            copy_idx(k0 + 1, dst1_v)
            start_gather(rows1_v)

            wait_gather(rows1_v)
            start_scatter(rows1_v, dst1_v, ssem1)
            wait_scatter(rows0_v, dst0_v, ssem0)
            copy_idx(k0 + 2, dst0_v)
            start_gather(rows0_v)
            return carry

        lax.fori_loop(0, NCHUNKS // 2, step, 0)

        wait_gather(rows0_v)
        start_scatter(rows0_v, dst0_v, ssem0)
        wait_scatter(rows1_v, dst1_v, ssem1)
        wait_scatter(rows0_v, dst0_v, ssem0)
        plsc.subcore_barrier()

        @pl.when(s < NZT)
        def _():
            pltpu.sync_copy(s_sh.at[pl.ds(r0, ZR)],
                            s_out.at[c, pl.ds(r0, ZR)])
            pltpu.sync_copy(g_sh.at[pl.ds(g0, GZR)],
                            g_out.at[c, pl.ds(g0, GZR)])

    return k(src, dst, q)


BLK = 1000


def _finish_body(q_ref, p_ref, s_ref, g_ref, obs_ref, gain_ref,
                 qo_ref, po_ref):
    q = q_ref[...]
    p = p_ref[...]
    s = (s_ref[0].astype(jnp.float32) + s_ref[1].astype(jnp.float32))
    deg = g_ref[0, :, 0:1] + g_ref[1, :, 0:1]
    msg = deg * q - s
    q_new = q + DT * p + (0.5 * DT * DT) * msg
    p_new = p + DT * msg

    @pl.when(pl.program_id(0) == 0)
    def _():
        innov = obs_ref[...] - q_new
        rows = lax.broadcasted_iota(jnp.int32, q_new.shape, 0)
        m = rows == 0
        qo_ref[...] = jnp.where(m, q_new + gain_ref[0:1] * innov, q_new)
        po_ref[...] = jnp.where(m, p_new + gain_ref[1:2] * innov, p_new)

    @pl.when(pl.program_id(0) != 0)
    def _():
        qo_ref[...] = q_new
        po_ref[...] = p_new


def _finish(q, p, s_parts, g_parts, obs, gain2):
    grid = (N // BLK,)
    return pl.pallas_call(
        _finish_body,
        grid=grid,
        in_specs=[
            pl.BlockSpec((BLK, D), lambda i: (i, 0)),
            pl.BlockSpec((BLK, D), lambda i: (i, 0)),
            pl.BlockSpec((NC, BLK, D), lambda i: (0, i, 0)),
            pl.BlockSpec((NC, BLK, GW), lambda i: (0, i, 0)),
            pl.BlockSpec((1, D), lambda i: (0, 0)),
            pl.BlockSpec((2, D), lambda i: (0, 0)),
        ],
        out_specs=[
            pl.BlockSpec((BLK, D), lambda i: (i, 0)),
            pl.BlockSpec((BLK, D), lambda i: (i, 0)),
        ],
        out_shape=[
            jax.ShapeDtypeStruct((N, D), jnp.float32),
            jax.ShapeDtypeStruct((N, D), jnp.float32),
        ],
    )(q, p, s_parts, g_parts, obs, gain2)


def kernel(node_q, node_p, edges, observations, kalman_gain):
    q = node_q.reshape(N, D)
    p = node_p.reshape(N, D)
    src = edges[:, 0]
    dst = edges[:, 1]
    s_parts, g_parts = _sc_scatter(src, dst, q.astype(jnp.bfloat16))
    obs = observations.reshape(1, D)
    gain2 = kalman_gain.reshape(2, D)
    qo, po = _finish(q, p, s_parts, g_parts, obs, gain2)
    return qo.reshape(node_q.shape), po.reshape(node_p.shape)


# speedup vs baseline: 15.8856x; 1.0208x over previous
import functools

import jax
import jax.numpy as jnp
from jax import lax
from jax.experimental import pallas as pl
from jax.experimental.pallas import tpu as pltpu
from jax.experimental.pallas import tpu_sc as plsc

N = 10000
D = 128
E = 320000
DT = 0.01

NC = 2
NS = 16
NW = NC * NS
EPW = E // NW
CHUNK = 400
NCHUNKS = EPW // CHUNK
ZR = 1000
NZT = N // ZR
BB = 40
NBB = ZR // BB
GW = 16
NPAD = 10240
GZR = NPAD // NZT
GB = 64
NGB = GZR // GB
PR = GZR // 8
PB = 64


def _sc_scatter(src, dst, q):
    mesh = plsc.VectorSubcoreMesh(core_axis_name="c", subcore_axis_name="s")

    @functools.partial(
        pl.kernel,
        out_type=(
            jax.ShapeDtypeStruct((NC, N, D), jnp.bfloat16),
            jax.ShapeDtypeStruct((NW, N), jnp.float32),
        ),
        mesh=mesh,
        scratch_types=[
            pltpu.VMEM((CHUNK,), jnp.int32),
            pltpu.VMEM((CHUNK,), jnp.int32),
            pltpu.VMEM((CHUNK,), jnp.int32),
            pltpu.VMEM((CHUNK, D), jnp.bfloat16),
            pltpu.VMEM((CHUNK, D), jnp.bfloat16),
            pltpu.VMEM((N,), jnp.float32),
            pltpu.VMEM((BB, D), jnp.bfloat16),
            pltpu.VMEM_SHARED((N, D), jnp.bfloat16),
            pltpu.SemaphoreType.DMA,
            pltpu.SemaphoreType.DMA,
            pltpu.SemaphoreType.DMA,
        ],
        compiler_params=pltpu.CompilerParams(use_tc_tiling_on_sc=False,
                                             needs_layout_passes=False),
    )
    def k(src_hbm, dst_hbm, q_hbm,
          s_out, g_out, src_v, dst0_v, dst1_v, rows0_v, rows1_v,
          deg_v, zb_v, s_sh, gsem, ssem0, ssem1):
        c = lax.axis_index("c")
        s = lax.axis_index("s")
        r0 = s * ZR

        zero16 = jnp.zeros((GW,), dtype=jnp.float32)
        one16 = jnp.full((GW,), 1.0, dtype=jnp.float32)
        zero32b = jnp.zeros((32,), dtype=jnp.bfloat16)
        for r in range(BB):
            for cc in range(D // 32):
                zb_v[r, pl.ds(cc * 32, 32)] = zero32b

        def zdeg(i, carry):
            deg_v[pl.ds(i * GW, GW)] = zero16
            return carry

        lax.fori_loop(0, N // GW, zdeg, 0)

        @pl.when(s < NZT)
        def _():
            for j in range(NBB):
                pltpu.sync_copy(zb_v, s_sh.at[pl.ds(r0 + j * BB, BB)])

        plsc.subcore_barrier()

        base = (c * NS + s) * EPW

        def copy_idx(k, dref):
            e0 = base + k * CHUNK
            pltpu.sync_copy(src_hbm.at[pl.ds(e0, CHUNK)], src_v)
            pltpu.sync_copy(dst_hbm.at[pl.ds(e0, CHUNK)], dref)

        def start_gather(rref):
            pltpu.async_copy(q_hbm.at[src_v], rref, gsem)

        def wait_gather(rref):
            pltpu.make_async_copy(q_hbm.at[src_v], rref, gsem).wait()

        def start_scatter(rref, dref, sem_):
            pltpu.async_copy(rref, s_sh.at[dref], sem_, add=True)

        def wait_scatter(rref, dref, sem_):
            pltpu.make_async_copy(rref, s_sh.at[dref], sem_).wait()

        def count_deg(dref):
            for g in range(CHUNK // GW):
                idx = dref[pl.ds(g * GW, GW)]
                plsc.addupdate_scatter(deg_v, [idx], one16)

        copy_idx(0, dst0_v)
        start_gather(rows0_v)

        def step(t, carry):
            k0 = 2 * t
            wait_gather(rows0_v)
            start_scatter(rows0_v, dst0_v, ssem0)

            @pl.when(t > 0)
            def _():
                wait_scatter(rows1_v, dst1_v, ssem1)

            copy_idx(k0 + 1, dst1_v)
            start_gather(rows1_v)
            count_deg(dst0_v)

            wait_gather(rows1_v)
            start_scatter(rows1_v, dst1_v, ssem1)
            wait_scatter(rows0_v, dst0_v, ssem0)
            copy_idx(k0 + 2, dst0_v)
            start_gather(rows0_v)
            count_deg(dst1_v)
            return carry

        lax.fori_loop(0, NCHUNKS // 2, step, 0)

        wait_gather(rows0_v)
        start_scatter(rows0_v, dst0_v, ssem0)
        count_deg(dst0_v)
        wait_scatter(rows1_v, dst1_v, ssem1)
        wait_scatter(rows0_v, dst0_v, ssem0)

        pltpu.sync_copy(deg_v, g_out.at[c * NS + s])
        plsc.subcore_barrier()

        @pl.when(s < NZT)
        def _():
            pltpu.sync_copy(s_sh.at[pl.ds(r0, ZR)],
                            s_out.at[c, pl.ds(r0, ZR)])

    return k(src, dst, q)


BLK = 1000


def _finish_body(q_ref, p_ref, s_ref, g_ref, obs_ref, gain_ref,
                 qo_ref, po_ref):
    q = q_ref[...]
    p = p_ref[...]
    s = (s_ref[0].astype(jnp.float32) + s_ref[1].astype(jnp.float32))
    deg = jnp.sum(g_ref[...], axis=1)[:, None]
    msg = deg * q - s
    q_new = q + DT * p + (0.5 * DT * DT) * msg
    p_new = p + DT * msg

    @pl.when(pl.program_id(0) == 0)
    def _():
        innov = obs_ref[...] - q_new
        rows = lax.broadcasted_iota(jnp.int32, q_new.shape, 0)
        m = rows == 0
        qo_ref[...] = jnp.where(m, q_new + gain_ref[0:1] * innov, q_new)
        po_ref[...] = jnp.where(m, p_new + gain_ref[1:2] * innov, p_new)

    @pl.when(pl.program_id(0) != 0)
    def _():
        qo_ref[...] = q_new
        po_ref[...] = p_new


def _finish(q, p, s_parts, g_parts, obs, gain2):
    grid = (N // BLK,)
    return pl.pallas_call(
        _finish_body,
        grid=grid,
        in_specs=[
            pl.BlockSpec((BLK, D), lambda i: (i, 0)),
            pl.BlockSpec((BLK, D), lambda i: (i, 0)),
            pl.BlockSpec((NC, BLK, D), lambda i: (0, i, 0)),
            pl.BlockSpec((BLK, NW), lambda i: (i, 0)),
            pl.BlockSpec((1, D), lambda i: (0, 0)),
            pl.BlockSpec((2, D), lambda i: (0, 0)),
        ],
        out_specs=[
            pl.BlockSpec((BLK, D), lambda i: (i, 0)),
            pl.BlockSpec((BLK, D), lambda i: (i, 0)),
        ],
        out_shape=[
            jax.ShapeDtypeStruct((N, D), jnp.float32),
            jax.ShapeDtypeStruct((N, D), jnp.float32),
        ],
    )(q, p, s_parts, g_parts, obs, gain2)


def kernel(node_q, node_p, edges, observations, kalman_gain):
    q = node_q.reshape(N, D)
    p = node_p.reshape(N, D)
    src = edges[:, 0]
    dst = edges[:, 1]
    s_parts, g_parts = _sc_scatter(src, dst, q.astype(jnp.bfloat16))
    g_t = g_parts.T
    obs = observations.reshape(1, D)
    gain2 = kalman_gain.reshape(2, D)
    qo, po = _finish(q, p, s_parts, g_t, obs, gain2)
    return qo.reshape(node_q.shape), po.reshape(node_p.shape)


# speedup vs baseline: 18.4935x; 1.1642x over previous
import functools

import jax
import jax.numpy as jnp
from jax import lax
from jax.experimental import pallas as pl
from jax.experimental.pallas import tpu as pltpu
from jax.experimental.pallas import tpu_sc as plsc

N = 10000
D = 128
E = 320000
DT = 0.01

NC = 2
NS = 16
NW = NC * NS
EPW = E // NW
CHUNK = 400
NCHUNKS = EPW // CHUNK
ZR = 1000
NZT = N // ZR
BB = 40
NBB = ZR // BB
GW = 16
NPAD = 10240
GZR = NPAD // NZT
GB = 64
NGB = GZR // GB
PR = GZR // 8
PB = 64


def _sc_scatter(src, dst, q):
    mesh = plsc.VectorSubcoreMesh(core_axis_name="c", subcore_axis_name="s")

    @functools.partial(
        pl.kernel,
        out_type=(
            jax.ShapeDtypeStruct((NC, N, D), jnp.bfloat16),
            jax.ShapeDtypeStruct((NW, N), jnp.float32),
        ),
        mesh=mesh,
        scratch_types=[
            pltpu.VMEM((CHUNK,), jnp.int32),
            pltpu.VMEM((CHUNK,), jnp.int32),
            pltpu.VMEM((CHUNK,), jnp.int32),
            pltpu.VMEM((CHUNK,), jnp.int32),
            pltpu.VMEM((CHUNK,), jnp.int32),
            pltpu.VMEM((CHUNK,), jnp.int32),
            pltpu.VMEM((CHUNK, D), jnp.bfloat16),
            pltpu.VMEM((CHUNK, D), jnp.bfloat16),
            pltpu.VMEM((CHUNK, D), jnp.bfloat16),
            pltpu.VMEM((N,), jnp.float32),
            pltpu.VMEM_SHARED((N, D), jnp.bfloat16),
            pltpu.SemaphoreType.DMA,
            pltpu.SemaphoreType.DMA,
            pltpu.SemaphoreType.DMA,
            pltpu.SemaphoreType.DMA,
            pltpu.SemaphoreType.DMA,
            pltpu.SemaphoreType.DMA,
        ],
        compiler_params=pltpu.CompilerParams(use_tc_tiling_on_sc=False,
                                             needs_layout_passes=False),
    )
    def k(src_hbm, dst_hbm, q_hbm,
          s_out, g_out, src0_v, src1_v, src2_v, dst0_v, dst1_v, dst2_v,
          rows0_v, rows1_v, rows2_v,
          deg_v, s_sh, gsem0, gsem1, gsem2, ssem0, ssem1, ssem2):
        c = lax.axis_index("c")
        s = lax.axis_index("s")
        r0 = s * ZR

        zero16 = jnp.zeros((GW,), dtype=jnp.float32)
        one16 = jnp.full((GW,), 1.0, dtype=jnp.float32)
        zero32b = jnp.zeros((32,), dtype=jnp.bfloat16)

        def zrow(i, carry):
            for cc in range(D // 32):
                rows0_v[i, pl.ds(cc * 32, 32)] = zero32b
            return carry

        lax.fori_loop(0, CHUNK, zrow, 0)

        def zdeg(i, carry):
            deg_v[pl.ds(i * GW, GW)] = zero16
            return carry

        lax.fori_loop(0, N // GW, zdeg, 0)

        @pl.when(s < NZT)
        def _():
            for j in range(ZR // CHUNK):
                pltpu.sync_copy(rows0_v,
                                s_sh.at[pl.ds(r0 + j * CHUNK, CHUNK)])
            rem = ZR - (ZR // CHUNK) * CHUNK
            if rem:
                pltpu.sync_copy(
                    rows0_v.at[pl.ds(0, rem)],
                    s_sh.at[pl.ds(r0 + (ZR // CHUNK) * CHUNK, rem)])

        plsc.subcore_barrier()

        base = (c * NS + s) * EPW

        srcs = (src0_v, src1_v, src2_v)
        dsts = (dst0_v, dst1_v, dst2_v)
        rows = (rows0_v, rows1_v, rows2_v)
        gsems = (gsem0, gsem1, gsem2)
        ssems = (ssem0, ssem1, ssem2)

        def copy_idx(k, b):
            e0 = base + k * CHUNK
            pltpu.sync_copy(src_hbm.at[pl.ds(e0, CHUNK)], srcs[b])
            pltpu.sync_copy(dst_hbm.at[pl.ds(e0, CHUNK)], dsts[b])

        def start_gather(b):
            pltpu.async_copy(q_hbm.at[srcs[b]], rows[b], gsems[b])

        def wait_gather(b):
            pltpu.make_async_copy(q_hbm.at[srcs[b]], rows[b],
                                  gsems[b]).wait()

        def start_scatter(b):
            pltpu.async_copy(rows[b], s_sh.at[dsts[b]], ssems[b], add=True)

        def wait_scatter(b):
            pltpu.make_async_copy(rows[b], s_sh.at[dsts[b]],
                                  ssems[b]).wait()

        def count_deg(b):
            for g in range(CHUNK // GW):
                idx = dsts[b][pl.ds(g * GW, GW)]
                plsc.addupdate_scatter(deg_v, [idx], one16)

        copy_idx(0, 0)
        start_gather(0)
        copy_idx(1, 1)
        start_gather(1)

        def step(t, carry):
            for j in range(3):
                b = j
                b2 = (j + 2) % 3
                wait_gather(b)
                start_scatter(b)
                if j == 0:
                    @pl.when(t > 0)
                    def _():
                        wait_scatter(b2)

                    copy_idx(3 * t + j + 2, b2)
                    start_gather(b2)
                elif j == 1:
                    wait_scatter(b2)
                    copy_idx(3 * t + j + 2, b2)
                    start_gather(b2)
                else:
                    @pl.when(t < NCHUNKS // 3 - 1)
                    def _():
                        wait_scatter(b2)
                        copy_idx(3 * t + j + 2, b2)
                        start_gather(b2)

                count_deg(b)
            return carry

        lax.fori_loop(0, NCHUNKS // 3, step, 0)

        b = (NCHUNKS - 1) % 3
        wait_gather(b)
        start_scatter(b)
        count_deg(b)
        wait_scatter((b + 1) % 3)
        wait_scatter((b + 2) % 3)
        wait_scatter(b)

        pltpu.sync_copy(deg_v, g_out.at[c * NS + s])
        plsc.subcore_barrier()

        @pl.when(s < NZT)
        def _():
            pltpu.sync_copy(s_sh.at[pl.ds(r0, ZR)],
                            s_out.at[c, pl.ds(r0, ZR)])

    return k(src, dst, q)


BLK = 1000


def _finish_body(q_ref, p_ref, s_ref, g_ref, obs_ref, gain_ref,
                 qo_ref, po_ref):
    q = q_ref[...]
    p = p_ref[...]
    s = (s_ref[0].astype(jnp.float32) + s_ref[1].astype(jnp.float32))
    deg = jnp.sum(g_ref[...], axis=1)[:, None]
    msg = deg * q - s
    q_new = q + DT * p + (0.5 * DT * DT) * msg
    p_new = p + DT * msg

    @pl.when(pl.program_id(0) == 0)
    def _():
        innov = obs_ref[...] - q_new
        rows = lax.broadcasted_iota(jnp.int32, q_new.shape, 0)
        m = rows == 0
        qo_ref[...] = jnp.where(m, q_new + gain_ref[0:1] * innov, q_new)
        po_ref[...] = jnp.where(m, p_new + gain_ref[1:2] * innov, p_new)

    @pl.when(pl.program_id(0) != 0)
    def _():
        qo_ref[...] = q_new
        po_ref[...] = p_new


def _finish(q, p, s_parts, g_parts, obs, gain2):
    grid = (N // BLK,)
    return pl.pallas_call(
        _finish_body,
        grid=grid,
        in_specs=[
            pl.BlockSpec((BLK, D), lambda i: (i, 0)),
            pl.BlockSpec((BLK, D), lambda i: (i, 0)),
            pl.BlockSpec((NC, BLK, D), lambda i: (0, i, 0)),
            pl.BlockSpec((BLK, NW), lambda i: (i, 0)),
            pl.BlockSpec((1, D), lambda i: (0, 0)),
            pl.BlockSpec((2, D), lambda i: (0, 0)),
        ],
        out_specs=[
            pl.BlockSpec((BLK, D), lambda i: (i, 0)),
            pl.BlockSpec((BLK, D), lambda i: (i, 0)),
        ],
        out_shape=[
            jax.ShapeDtypeStruct((N, D), jnp.float32),
            jax.ShapeDtypeStruct((N, D), jnp.float32),
        ],
    )(q, p, s_parts, g_parts, obs, gain2)


def kernel(node_q, node_p, edges, observations, kalman_gain):
    q = node_q.reshape(N, D)
    p = node_p.reshape(N, D)
    src = edges[:, 0]
    dst = edges[:, 1]
    s_parts, g_parts = _sc_scatter(src, dst, q.astype(jnp.bfloat16))
    g_t = g_parts.T
    obs = observations.reshape(1, D)
    gain2 = kalman_gain.reshape(2, D)
    qo, po = _finish(q, p, s_parts, g_t, obs, gain2)
    return qo.reshape(node_q.shape), po.reshape(node_p.shape)
